# baseline (speedup 1.0000x reference)
import functools

import jax
import jax.numpy as jnp
from jax import lax
from jax.experimental import pallas as pl
from jax.experimental.pallas import tpu as pltpu
from jax.experimental.pallas import tpu_sc as plsc

N, L, HALF = 16, 2048, 64
A = 14
DF = 2 * HALF
R = 21

NW = 32
RES_PER_W = N * L // NW
ROWS_PER_W = RES_PER_W * A
PC = 128
NPC = RES_PER_W // PC
PGRP = PC * A // 16


def _build_table2_kernel(rt_ref, at_ref, out_ref):
    rt = rt_ref[...]
    at = at_ref[...]
    out_ref[...] = jnp.concatenate(
        [
            jnp.broadcast_to(rt[:, None, :], (R, A, HALF)),
            jnp.broadcast_to(at[None, :, :], (R, A, HALF)),
        ],
        axis=-1,
    )


_GATHER_DNUMS = lax.GatherDimensionNumbers(
    offset_dims=(), collapsed_slice_dims=(0,), start_index_map=(0,))


def _vgather(v, idx):
    return lax.gather(
        v, idx[:, None], _GATHER_DNUMS, (1,),
        mode=lax.GatherScatterMode.PROMISE_IN_BOUNDS)


def _make_expand_kernel():
    mesh = plsc.VectorSubcoreMesh(core_axis_name="c", subcore_axis_name="s")

    @functools.partial(
        pl.kernel,
        mesh=mesh,
        out_type=(
            jax.ShapeDtypeStruct((N, L * A, DF), jnp.float32),
            jax.ShapeDtypeStruct((3, N, L * A), jnp.float32),
        ),
        compiler_params=pltpu.CompilerParams(use_tc_tiling_on_sc=False),
        scratch_types=[
            pltpu.VMEM((RES_PER_W,), jnp.int32),
            pltpu.VMEM((R * A, DF), jnp.float32),
            pltpu.VMEM((PC * A * 3,), jnp.float32),
            pltpu.VMEM((PC * A * 3,), jnp.float32),
            pltpu.VMEM((3 * ROWS_PER_W,), jnp.float32),
            pltpu.SemaphoreType.DMA,
            pltpu.SemaphoreType.DMA,
            pltpu.SemaphoreType.DMA,
        ],
    )
    def expand_kernel(aa_hbm, pos_hbm, table_hbm, out_hbm, coors_hbm,
                      aa_v, table_v, pos_v0, pos_v1, cbuf,
                      osem, psem, csem):
        pos_vs = (pos_v0, pos_v1)
        wid = lax.axis_index("s") * 2 + lax.axis_index("c")
        n = wid // 2
        half = wid % 2
        off = half * ROWS_PER_W
        l0 = half * RES_PER_W
        pltpu.sync_copy(table_hbm, table_v)
        pltpu.sync_copy(aa_hbm.at[n, pl.ds(l0, RES_PER_W)], aa_v)

        lanes = lax.iota(jnp.int32, 16)

        def stage_pos(c, s):
            pltpu.async_copy(
                pos_hbm.at[n, pl.ds((l0 + c * PC) * A * 3, PC * A * 3)],
                pos_vs[s], psem)

        def wait_pos(c, s):
            pltpu.make_async_copy(
                pos_hbm.at[n, pl.ds((l0 + c * PC) * A * 3, PC * A * 3)],
                pos_vs[s], psem).wait()

        stage_pos(0, 0)

        for c in range(NPC):
            s = c % 2
            wait_pos(c, s)
            if c + 1 < NPC:
                stage_pos(c + 1, 1 - s)

            def grp_body(g, carry2, s=s, c=c):
                v0 = pos_vs[s][pl.ds(g * 48, 16)]
                v1 = pos_vs[s][pl.ds(g * 48 + 16, 16)]
                v2 = pos_vs[s][pl.ds(g * 48 + 32, 16)]
                pos = c * PC * A + g * 16
                for d in range(3):
                    t = 3 * lanes + d
                    li = t & 15
                    sel = t >> 4
                    a0 = _vgather(v0, li)
                    a1 = _vgather(v1, li)
                    a2 = _vgather(v2, li)
                    vals = jnp.where(sel == 0, a0,
                                     jnp.where(sel == 1, a1, a2))
                    cbuf[pl.ds(d * ROWS_PER_W + pos, 16)] = vals
                return carry2

            lax.fori_loop(0, PGRP, grp_body, 0)
        for d in range(3):
            pltpu.async_copy(
                cbuf.at[pl.ds(d * ROWS_PER_W, ROWS_PER_W)],
                coors_hbm.at[d, n, pl.ds(off, ROWS_PER_W)], csem)

        def issue_group(g, carry):
            rows16 = aa_v[pl.ds(g * 16, 16)]
            base = off + g * 16 * A
            for k in range(16):
                row = rows16[k]
                pltpu.async_copy(
                    table_v.at[pl.ds(row * A, A)],
                    out_hbm.at[n, pl.ds(base + k * A, A)],
                    osem)
            return carry

        lax.fori_loop(0, RES_PER_W // 16, issue_group, 0)

        def drain(r, carry):
            pltpu.make_async_copy(
                table_v.at[pl.ds(0, A)],
                out_hbm.at[n, pl.ds(off + r * A, A)],
                osem).wait()
            return carry

        lax.fori_loop(0, RES_PER_W, drain, 0)
        for d in range(3):
            pltpu.make_async_copy(
                cbuf.at[pl.ds(d * ROWS_PER_W, ROWS_PER_W)],
                coors_hbm.at[d, n, pl.ds(off, ROWS_PER_W)], csem).wait()

    return expand_kernel


_EXPAND = _make_expand_kernel()


def kernel(aa, pos14, atom_mask, residual_table, atom_table):
    table3 = pl.pallas_call(
        _build_table2_kernel,
        out_shape=jax.ShapeDtypeStruct((R, A, DF), jnp.float32),
    )(residual_table, atom_table)
    table2 = table3.reshape(R * A, DF)
    feats, coors3 = _EXPAND(aa.astype(jnp.int32),
                            pos14.reshape(N, L * A * 3), table2)
    coors = coors3.transpose(1, 2, 0)
    mask = atom_mask.reshape(N, L * A)
    return (feats, coors, mask)


# speedup vs baseline: 1.4908x; 1.4908x over previous
import functools

import jax
import jax.numpy as jnp
from jax import lax
from jax.experimental import pallas as pl
from jax.experimental.pallas import tpu as pltpu
from jax.experimental.pallas import tpu_sc as plsc

N, L, HALF = 16, 2048, 64
A = 14
DF = 2 * HALF
R = 21

NW = 32
RES_PER_W = N * L // NW
ROWS_PER_W = RES_PER_W * A


def _build_table2_kernel(rt_ref, at_ref, out_ref):
    rt = rt_ref[...]
    at = at_ref[...]
    out_ref[...] = jnp.concatenate(
        [
            jnp.broadcast_to(rt[:, None, :], (R, A, HALF)),
            jnp.broadcast_to(at[None, :, :], (R, A, HALF)),
        ],
        axis=-1,
    )


def _make_expand_kernel():
    mesh = plsc.VectorSubcoreMesh(core_axis_name="c", subcore_axis_name="s")

    @functools.partial(
        pl.kernel,
        mesh=mesh,
        out_type=jax.ShapeDtypeStruct((N, L * A, DF), jnp.float32),
        compiler_params=pltpu.CompilerParams(use_tc_tiling_on_sc=False),
        scratch_types=[
            pltpu.VMEM((RES_PER_W,), jnp.int32),
            pltpu.VMEM((R * A, DF), jnp.float32),
            pltpu.SemaphoreType.DMA,
        ],
    )
    def expand_kernel(aa_hbm, table_hbm, out_hbm, aa_v, table_v, osem):
        wid = lax.axis_index("s") * 2 + lax.axis_index("c")
        n = wid // 2
        off = (wid % 2) * ROWS_PER_W
        pltpu.sync_copy(table_hbm, table_v)
        pltpu.sync_copy(aa_hbm.at[n, pl.ds((wid % 2) * RES_PER_W, RES_PER_W)],
                        aa_v)

        def issue_group(g, carry):
            rows16 = aa_v[pl.ds(g * 16, 16)]
            base = off + g * 16 * A
            for k in range(16):
                row = rows16[k]
                pltpu.async_copy(
                    table_v.at[pl.ds(row * A, A)],
                    out_hbm.at[n, pl.ds(base + k * A, A)],
                    osem)
            return carry

        lax.fori_loop(0, RES_PER_W // 16, issue_group, 0)

        def drain(r, carry):
            pltpu.make_async_copy(
                table_v.at[pl.ds(0, A)],
                out_hbm.at[n, pl.ds(off + r * A, A)],
                osem).wait()
            return carry

        lax.fori_loop(0, RES_PER_W, drain, 0)

    return expand_kernel


_EXPAND = _make_expand_kernel()


def kernel(aa, pos14, atom_mask, residual_table, atom_table):
    table3 = pl.pallas_call(
        _build_table2_kernel,
        out_shape=jax.ShapeDtypeStruct((R, A, DF), jnp.float32),
    )(residual_table, atom_table)
    table2 = table3.reshape(R * A, DF)
    feats = _EXPAND(aa.astype(jnp.int32), table2)
    coors = jnp.stack([pos14[:, :, :, d].reshape(N, L * A)
                       for d in range(3)], axis=-1)
    mask = atom_mask.reshape(N, L * A)
    return (feats, coors, mask)
